# Initial kernel scaffold; baseline (speedup 1.0000x reference)
import jax
import jax.numpy as jnp
from jax.experimental import pallas as pl


def kernel(patch, patch_lib):
    raise NotImplementedError("write your pallas kernel here")


# speedup vs baseline: 5.4597x; 5.4597x over previous
import numpy as np
import jax
import jax.numpy as jnp
from jax import lax
from jax.experimental import pallas as pl
from jax.experimental.pallas import tpu as pltpu

_IMG = 224
_FMAP = 26
_SIGMA = 4.0
_NQ = 676
_MP = 704
_BN = 1024


def _resize_blur_matrix() -> np.ndarray:
    scale = _IMG / _FMAP
    x = (np.arange(_IMG) + 0.5) / scale - 0.5
    j = np.arange(_FMAP)
    w = np.maximum(0.0, 1.0 - np.abs(x[:, None] - j[None, :]))
    w = w / w.sum(axis=1, keepdims=True)
    ks = 2 * int(4.0 * _SIGMA) + 1
    r = ks // 2
    t = np.arange(ks) - r
    k = np.exp(-0.5 * (t / _SIGMA) ** 2)
    k = k / k.sum()
    g = np.zeros((_IMG, _IMG))
    for i in range(_IMG):
        for dt in range(ks):
            p = i + dt - r
            if p < 0:
                p = -p
            if p > _IMG - 1:
                p = 2 * (_IMG - 1) - p
            g[i, p] += k[dt]
    return (g @ w).astype(np.float32)


_A = _resize_blur_matrix()
_AT = np.ascontiguousarray(_A.T)


def _k1_body(patch_ref, lib_ref, minval_ref, sstar_ref, sidx_ref, staridx_ref,
             curmin_ref, curidx_ref):
    j = pl.program_id(0)
    nb = pl.num_programs(0)
    a = patch_ref[...]
    b = lib_ref[...]
    ab = lax.dot_general(b.astype(jnp.bfloat16), a.astype(jnp.bfloat16),
                         (((1,), (1,)), ((), ())),
                         preferred_element_type=jnp.float32)
    b2 = jnp.sum(b * b, axis=1, keepdims=True)
    d2 = b2 - 2.0 * ab
    bmin = jnp.min(d2, axis=0, keepdims=True)
    rows = lax.broadcasted_iota(jnp.int32, d2.shape, 0)
    bidx = jnp.min(jnp.where(d2 == bmin, rows, d2.shape[0]),
                   axis=0, keepdims=True)

    @pl.when(j == 0)
    def _init():
        curmin_ref[...] = jnp.full(curmin_ref.shape, jnp.inf, jnp.float32)
        curidx_ref[...] = jnp.zeros(curidx_ref.shape, jnp.int32)

    cm = curmin_ref[...]
    better = bmin < cm
    curmin_ref[...] = jnp.where(better, bmin, cm)
    curidx_ref[...] = jnp.where(better, j * _BN + bidx, curidx_ref[...])

    @pl.when(j == nb - 1)
    def _fin():
        ones = jnp.ones((1, a.shape[1]), jnp.float32)
        a2 = lax.dot_general(ones, a * a, (((1,), (1,)), ((), ())),
                             preferred_element_type=jnp.float32)
        dist = jnp.sqrt(jnp.maximum(curmin_ref[...] + a2, 1e-12))
        minval_ref[...] = dist
        cols = lax.broadcasted_iota(jnp.int32, dist.shape, 1)
        mv = jnp.where(cols < _NQ, dist, -1.0)
        smax = jnp.max(mv)
        sidx = jnp.min(jnp.where(mv == smax, cols, _MP))
        sstar_ref[0, 0] = smax
        sidx_ref[0, 0] = sidx
        staridx_ref[0, 0] = jnp.sum(
            jnp.where(cols == sidx, curidx_ref[...], 0))


def _k2_body(m8_ref, sstar_ref, lib_ref, s_ref, d2all_ref):
    j = pl.program_id(0)
    nb = pl.num_programs(0)
    m = m8_ref[...]
    b = lib_ref[...]
    ab = lax.dot_general(m.astype(jnp.bfloat16), b.astype(jnp.bfloat16),
                         (((1,), (1,)), ((), ())),
                         preferred_element_type=jnp.float32)
    ones = jnp.ones((1, b.shape[1]), jnp.float32)
    b2 = lax.dot_general(ones, b * b, (((1,), (1,)), ((), ())),
                         preferred_element_type=jnp.float32)
    d2all_ref[:, pl.ds(j * _BN, _BN)] = b2 - 2.0 * ab

    @pl.when(j == nb - 1)
    def _fin():
        n = d2all_ref.shape[1]
        m2 = jnp.sum(m * m, axis=1, keepdims=True)
        all_d2 = d2all_ref[...] + m2
        star = all_d2[0:1, :]
        test = all_d2[1:2, :]
        cols = lax.broadcasted_iota(jnp.int32, star.shape, 1)
        inf = jnp.float32(jnp.inf)
        i0 = jnp.min(jnp.where(star == jnp.min(star), cols, n))
        star1 = jnp.where(cols == i0, inf, star)
        i1 = jnp.min(jnp.where(star1 == jnp.min(star1), cols, n))
        star2 = jnp.where(cols == i1, inf, star1)
        i2 = jnp.min(jnp.where(star2 == jnp.min(star2), cols, n))
        dk1 = jnp.sqrt(jnp.maximum(
            jnp.sum(jnp.where(cols == i1, test, 0.0)), 0.0))
        dk2 = jnp.sqrt(jnp.maximum(
            jnp.sum(jnp.where(cols == i2, test, 0.0)), 0.0))
        dcap = jnp.sqrt(jnp.float32(m.shape[1]))
        sstar = sstar_ref[0, 0]
        w = 1.0 - jnp.exp(sstar / dcap) / (jnp.exp(dk1 / dcap)
                                           + jnp.exp(dk2 / dcap))
        s_ref[0, 0] = w * sstar


def _k3_body(s_ref, a_ref, at_ref, out_ref):
    t = lax.dot_general(a_ref[...], s_ref[...], (((1,), (0,)), ((), ())),
                        preferred_element_type=jnp.float32)
    out_ref[...] = lax.dot_general(t, at_ref[...], (((1,), (0,)), ((), ())),
                                   preferred_element_type=jnp.float32)


def kernel(patch, patch_lib):
    n, d = patch_lib.shape
    nb = n // _BN
    patch_p = jnp.pad(patch, ((0, _MP - patch.shape[0]), (0, 0)))

    minval, sstar, sidx, staridx = pl.pallas_call(
        _k1_body,
        grid=(nb,),
        in_specs=[
            pl.BlockSpec((_MP, d), lambda j: (0, 0)),
            pl.BlockSpec((_BN, d), lambda j: (j, 0)),
        ],
        out_specs=[
            pl.BlockSpec((1, _MP), lambda j: (0, 0)),
            pl.BlockSpec(memory_space=pltpu.SMEM),
            pl.BlockSpec(memory_space=pltpu.SMEM),
            pl.BlockSpec(memory_space=pltpu.SMEM),
        ],
        out_shape=[
            jax.ShapeDtypeStruct((1, _MP), jnp.float32),
            jax.ShapeDtypeStruct((1, 1), jnp.float32),
            jax.ShapeDtypeStruct((1, 1), jnp.int32),
            jax.ShapeDtypeStruct((1, 1), jnp.int32),
        ],
        scratch_shapes=[
            pltpu.VMEM((1, _MP), jnp.float32),
            pltpu.VMEM((1, _MP), jnp.int32),
        ],
    )(patch_p, patch_lib)

    m_star = lax.dynamic_index_in_dim(patch_lib, staridx[0, 0], axis=0)
    m_test = lax.dynamic_index_in_dim(patch, sidx[0, 0], axis=0)
    m8 = jnp.zeros((8, d), jnp.float32)
    m8 = lax.dynamic_update_slice(m8, m_star, (0, 0))
    m8 = lax.dynamic_update_slice(m8, m_test, (1, 0))

    s = pl.pallas_call(
        _k2_body,
        grid=(nb,),
        in_specs=[
            pl.BlockSpec((8, d), lambda j: (0, 0)),
            pl.BlockSpec(memory_space=pltpu.SMEM),
            pl.BlockSpec((_BN, d), lambda j: (j, 0)),
        ],
        out_specs=pl.BlockSpec(memory_space=pltpu.SMEM),
        out_shape=jax.ShapeDtypeStruct((1, 1), jnp.float32),
        scratch_shapes=[pltpu.VMEM((8, n), jnp.float32)],
    )(m8, sstar, patch_lib)

    smat = minval[0, :_NQ].reshape(_FMAP, _FMAP)
    smap = pl.pallas_call(
        _k3_body,
        in_specs=[
            pl.BlockSpec((_FMAP, _FMAP), lambda: (0, 0)),
            pl.BlockSpec((_IMG, _FMAP), lambda: (0, 0)),
            pl.BlockSpec((_FMAP, _IMG), lambda: (0, 0)),
        ],
        out_specs=pl.BlockSpec((_IMG, _IMG), lambda: (0, 0)),
        out_shape=jax.ShapeDtypeStruct((_IMG, _IMG), jnp.float32),
    )(smat, jnp.asarray(_A), jnp.asarray(_AT))

    return s.reshape(()), smap.reshape(1, 1, _IMG, _IMG)
